# Initial kernel scaffold; baseline (speedup 1.0000x reference)
import jax
import jax.numpy as jnp
from jax.experimental import pallas as pl


def kernel(x, edge_index, W1, b1, W2, b2, Wmu, bmu, Wls, bls):
    raise NotImplementedError("write your pallas kernel here")


# speedup vs baseline: 14.8879x; 14.8879x over previous
import functools

import jax
import jax.numpy as jnp
from jax import lax
from jax.experimental import pallas as pl
from jax.experimental.pallas import tpu as pltpu
from jax.experimental.pallas import tpu_sc as plsc

N = 10000
F = 128
NC = 2
NS = 16
NW = NC * NS
CHUNK = 128
NCHUNK = 80
EPT = NCHUNK * CHUNK
EPAD = NW * EPT
NPAD = 10240
STRIPE = NPAD // NS
BR = 1280
GRID = NPAD // BR

_mesh = plsc.VectorSubcoreMesh(core_axis_name="c", subcore_axis_name="s")


@functools.partial(
    pl.kernel,
    out_type=jax.ShapeDtypeStruct((NC, NPAD, F), jnp.float32),
    mesh=_mesh,
    scratch_types=[
        pltpu.VMEM((CHUNK,), jnp.int32),
        pltpu.VMEM((CHUNK, F), jnp.float32),
        pltpu.VMEM_SHARED((NPAD, F), jnp.float32),
    ],
)
def _deg_pass(dst_hbm, eye_hbm, z_hbm, out_hbm, idx_v, upd_v, deg_sh):
    c = lax.axis_index("c")
    s = lax.axis_index("s")
    wid = s * NC + c
    pltpu.sync_copy(eye_hbm, upd_v)
    pltpu.sync_copy(z_hbm.at[pl.ds(s * STRIPE, STRIPE)],
                    deg_sh.at[pl.ds(s * STRIPE, STRIPE)])
    plsc.subcore_barrier()

    def body(j, carry):
        pltpu.sync_copy(dst_hbm.at[wid, j], idx_v)
        pltpu.sync_copy(upd_v, deg_sh.at[idx_v], add=True)
        return carry

    lax.fori_loop(0, NCHUNK, body, 0)
    plsc.subcore_barrier()
    pltpu.sync_copy(deg_sh.at[pl.ds(s * STRIPE, STRIPE)],
                    out_hbm.at[c, pl.ds(s * STRIPE, STRIPE)])


@functools.partial(
    pl.kernel,
    out_type=jax.ShapeDtypeStruct((NC, NPAD, F), jnp.float32),
    mesh=_mesh,
    scratch_types=[
        pltpu.VMEM((CHUNK,), jnp.int32),
        pltpu.VMEM((CHUNK,), jnp.int32),
        pltpu.VMEM((CHUNK, F), jnp.float32),
        pltpu.VMEM_SHARED((NPAD, F), jnp.float32),
        pltpu.SemaphoreType.DMA,
    ],
)
def _agg_pass(src_hbm, dst_hbm, hs_hbm, z_hbm, out_hbm,
              sidx_v, didx_v, rows_v, acc_sh, sem):
    c = lax.axis_index("c")
    s = lax.axis_index("s")
    wid = s * NC + c
    pltpu.sync_copy(z_hbm.at[pl.ds(s * STRIPE, STRIPE)],
                    acc_sh.at[pl.ds(s * STRIPE, STRIPE)])
    plsc.subcore_barrier()

    def body(j, carry):
        pltpu.sync_copy(src_hbm.at[wid, j], sidx_v)
        pltpu.sync_copy(dst_hbm.at[wid, j], didx_v)
        pltpu.async_copy(hs_hbm.at[sidx_v], rows_v, sem).wait()
        pltpu.sync_copy(rows_v, acc_sh.at[didx_v], add=True)
        return carry

    lax.fori_loop(0, NCHUNK, body, 0)
    plsc.subcore_barrier()
    pltpu.sync_copy(acc_sh.at[pl.ds(s * STRIPE, STRIPE)],
                    out_hbm.at[c, pl.ds(s * STRIPE, STRIPE)])


def _dinv(degp):
    return lax.rsqrt(1.0 + degp[0, :, 0:1] + degp[1, :, 0:1])


def _k1_body(degp, x, w, b, o):
    dinv = _dinv(degp)
    lin = jnp.dot(x[...], w[...], preferred_element_type=jnp.float32)
    o[...] = (lin + b[...][None, :]) * dinv


def _k23_body(degp, a, hs, w, b, o):
    dinv = _dinv(degp)
    h = jnp.maximum((a[0] + a[1] + hs[...]) * dinv, 0.0)
    lin = jnp.dot(h, w[...], preferred_element_type=jnp.float32)
    o[...] = (lin + b[...][None, :]) * dinv


def _k4_body(degp, a, hs, o):
    o[...] = (a[0] + a[1] + hs[...]) * _dinv(degp)


_spec_degp = pl.BlockSpec((NC, BR, F), lambda i: (0, i, 0))
_spec_rows = pl.BlockSpec((BR, F), lambda i: (i, 0))
_spec_acc = pl.BlockSpec((NC, BR, F), lambda i: (0, i, 0))
_spec_w = pl.BlockSpec((F, F), lambda i: (0, 0))
_spec_b = pl.BlockSpec((F,), lambda i: (0,))
_out_rows = jax.ShapeDtypeStruct((NPAD, F), jnp.float32)

_k1 = pl.pallas_call(
    _k1_body, grid=(GRID,),
    in_specs=[_spec_degp, _spec_rows, _spec_w, _spec_b],
    out_specs=_spec_rows, out_shape=_out_rows)

_k23 = pl.pallas_call(
    _k23_body, grid=(GRID,),
    in_specs=[_spec_degp, _spec_acc, _spec_rows, _spec_w, _spec_b],
    out_specs=_spec_rows, out_shape=_out_rows)

_k4 = pl.pallas_call(
    _k4_body, grid=(GRID,),
    in_specs=[_spec_degp, _spec_acc, _spec_rows],
    out_specs=_spec_rows, out_shape=_out_rows)


def kernel(x, edge_index, W1, b1, W2, b2, Wmu, bmu, Wls, bls):
    n, f = x.shape
    e = edge_index.shape[1]
    src = edge_index[0].astype(jnp.int32)
    dst = edge_index[1].astype(jnp.int32)
    pad = EPAD - e
    ar = jnp.arange(pad, dtype=jnp.int32)
    pad_src = (ar * 131) % n
    pad_dst = n + ar % (NPAD - n)
    src_r = jnp.concatenate([src, pad_src]).reshape(NW, NCHUNK, CHUNK)
    dst_r = jnp.concatenate([dst, pad_dst]).reshape(NW, NCHUNK, CHUNK)
    xp = jnp.pad(x, ((0, NPAD - n), (0, 0)))
    Wml = jnp.concatenate([Wmu, Wls], axis=1)
    bml = jnp.concatenate([bmu, bls])
    zini = jnp.zeros((NPAD, F), jnp.float32)
    eye0 = jnp.zeros((CHUNK, F), jnp.float32).at[:, 0].set(1.0)

    degp = _deg_pass(dst_r, eye0, zini)

    hs1 = _k1(degp, xp, W1, b1)
    a1 = _agg_pass(src_r, dst_r, hs1, zini)
    hs2 = _k23(degp, a1, hs1, W2, b2)
    a2 = _agg_pass(src_r, dst_r, hs2, zini)
    hs3 = _k23(degp, a2, hs2, Wml, bml)
    a3 = _agg_pass(src_r, dst_r, hs3, zini)
    out = _k4(degp, a3, hs3)
    return out[:n, :64], out[:n, 64:]


# speedup vs baseline: 26.6256x; 1.7884x over previous
import functools

import jax
import jax.numpy as jnp
from jax import lax
from jax.experimental import pallas as pl
from jax.experimental.pallas import tpu as pltpu
from jax.experimental.pallas import tpu_sc as plsc

N = 10000
F = 128
NC = 2
NS = 16
NW = NC * NS
CHUNK = 128
NCHUNK = 80
EPT = NCHUNK * CHUNK
EPAD = NW * EPT
NPAD = 10240
STRIPE = NPAD // NS
BR = 1280
GRID = NPAD // BR

_mesh = plsc.VectorSubcoreMesh(core_axis_name="c", subcore_axis_name="s")


@functools.partial(
    pl.kernel,
    out_type=jax.ShapeDtypeStruct((NC, NPAD, F), jnp.float32),
    mesh=_mesh,
    scratch_types=[
        pltpu.VMEM((CHUNK,), jnp.int32),
        pltpu.VMEM((CHUNK, F), jnp.float32),
        pltpu.VMEM_SHARED((NPAD, F), jnp.float32),
    ],
)
def _deg_pass(dst_hbm, eye_hbm, z_hbm, out_hbm, idx_v, upd_v, deg_sh):
    c = lax.axis_index("c")
    s = lax.axis_index("s")
    wid = s * NC + c
    pltpu.sync_copy(eye_hbm, upd_v)
    pltpu.sync_copy(z_hbm.at[pl.ds(s * STRIPE, STRIPE)],
                    deg_sh.at[pl.ds(s * STRIPE, STRIPE)])
    plsc.subcore_barrier()

    def body(j, carry):
        pltpu.sync_copy(dst_hbm.at[wid, j], idx_v)
        pltpu.sync_copy(upd_v, deg_sh.at[idx_v], add=True)
        return carry

    lax.fori_loop(0, NCHUNK, body, 0)
    plsc.subcore_barrier()
    pltpu.sync_copy(deg_sh.at[pl.ds(s * STRIPE, STRIPE)],
                    out_hbm.at[c, pl.ds(s * STRIPE, STRIPE)])


@functools.partial(
    pl.kernel,
    out_type=jax.ShapeDtypeStruct((NC, NPAD, F), jnp.float32),
    mesh=_mesh,
    scratch_types=[
        pltpu.VMEM((NCHUNK, CHUNK), jnp.int32),
        pltpu.VMEM((CHUNK,), jnp.int32),
        pltpu.VMEM((CHUNK,), jnp.int32),
        pltpu.VMEM((CHUNK, F), jnp.float32),
        pltpu.VMEM((CHUNK, F), jnp.float32),
        pltpu.VMEM_SHARED((NPAD, F), jnp.float32),
        pltpu.SemaphoreType.DMA,
        pltpu.SemaphoreType.DMA,
        pltpu.SemaphoreType.DMA,
        pltpu.SemaphoreType.DMA,
    ],
)
def _agg_pass(src_hbm, dst_hbm, hs_hbm, z_hbm, out_hbm,
              src_v, didx0, didx1, rows0, rows1, acc_sh,
              semg0, semg1, semi0, semi1):
    c = lax.axis_index("c")
    s = lax.axis_index("s")
    wid = s * NC + c
    pltpu.sync_copy(src_hbm.at[wid], src_v)
    pltpu.sync_copy(z_hbm.at[pl.ds(s * STRIPE, STRIPE)],
                    acc_sh.at[pl.ds(s * STRIPE, STRIPE)])
    plsc.subcore_barrier()

    didx = (didx0, didx1)
    rows = (rows0, rows1)
    semg = (semg0, semg1)
    semi = (semi0, semi1)

    def issue(j, b):
        pltpu.async_copy(dst_hbm.at[wid, j], didx[b], semi[b])
        pltpu.async_copy(hs_hbm.at[src_v.at[j]], rows[b], semg[b])

    def wait_and_scatter(j, b):
        pltpu.make_async_copy(dst_hbm.at[wid, j], didx[b], semi[b]).wait()
        pltpu.make_async_copy(hs_hbm.at[src_v.at[j]], rows[b], semg[b]).wait()
        pltpu.sync_copy(rows[b], acc_sh.at[didx[b]], add=True)

    issue(0, 0)

    def body(jj, carry):
        j0 = jj * 2
        issue(j0 + 1, 1)
        wait_and_scatter(j0, 0)

        @pl.when(j0 + 2 < NCHUNK)
        def _():
            issue(j0 + 2, 0)

        wait_and_scatter(j0 + 1, 1)
        return carry

    lax.fori_loop(0, NCHUNK // 2, body, 0)
    plsc.subcore_barrier()
    pltpu.sync_copy(acc_sh.at[pl.ds(s * STRIPE, STRIPE)],
                    out_hbm.at[c, pl.ds(s * STRIPE, STRIPE)])


def _dinv(degp):
    return lax.rsqrt(1.0 + degp[0, :, 0:1] + degp[1, :, 0:1])


def _k1_body(degp, x, w, b, o):
    dinv = _dinv(degp)
    lin = jnp.dot(x[...], w[...], preferred_element_type=jnp.float32)
    o[...] = (lin + b[...][None, :]) * dinv


def _k23_body(degp, a, hs, w, b, o):
    dinv = _dinv(degp)
    h = jnp.maximum((a[0] + a[1] + hs[...]) * dinv, 0.0)
    lin = jnp.dot(h, w[...], preferred_element_type=jnp.float32)
    o[...] = (lin + b[...][None, :]) * dinv


def _k4_body(degp, a, hs, o):
    o[...] = (a[0] + a[1] + hs[...]) * _dinv(degp)


_spec_degp = pl.BlockSpec((NC, BR, F), lambda i: (0, i, 0))
_spec_rows = pl.BlockSpec((BR, F), lambda i: (i, 0))
_spec_acc = pl.BlockSpec((NC, BR, F), lambda i: (0, i, 0))
_spec_w = pl.BlockSpec((F, F), lambda i: (0, 0))
_spec_b = pl.BlockSpec((F,), lambda i: (0,))
_out_rows = jax.ShapeDtypeStruct((NPAD, F), jnp.float32)

_k1 = pl.pallas_call(
    _k1_body, grid=(GRID,),
    in_specs=[_spec_degp, _spec_rows, _spec_w, _spec_b],
    out_specs=_spec_rows, out_shape=_out_rows)

_k23 = pl.pallas_call(
    _k23_body, grid=(GRID,),
    in_specs=[_spec_degp, _spec_acc, _spec_rows, _spec_w, _spec_b],
    out_specs=_spec_rows, out_shape=_out_rows)

_k4 = pl.pallas_call(
    _k4_body, grid=(GRID,),
    in_specs=[_spec_degp, _spec_acc, _spec_rows],
    out_specs=_spec_rows, out_shape=_out_rows)


def kernel(x, edge_index, W1, b1, W2, b2, Wmu, bmu, Wls, bls):
    n, f = x.shape
    e = edge_index.shape[1]
    src = edge_index[0].astype(jnp.int32)
    dst = edge_index[1].astype(jnp.int32)
    pad = EPAD - e
    ar = jnp.arange(pad, dtype=jnp.int32)
    pad_src = (ar * 131) % n
    pad_dst = n + ar % (NPAD - n)
    src_r = jnp.concatenate([src, pad_src]).reshape(NW, NCHUNK, CHUNK)
    dst_r = jnp.concatenate([dst, pad_dst]).reshape(NW, NCHUNK, CHUNK)
    xp = jnp.pad(x, ((0, NPAD - n), (0, 0)))
    Wml = jnp.concatenate([Wmu, Wls], axis=1)
    bml = jnp.concatenate([bmu, bls])
    zini = jnp.zeros((NPAD, F), jnp.float32)
    eye0 = jnp.zeros((CHUNK, F), jnp.float32).at[:, 0].set(1.0)

    degp = _deg_pass(dst_r, eye0, zini)

    hs1 = _k1(degp, xp, W1, b1)
    a1 = _agg_pass(src_r, dst_r, hs1, zini)
    hs2 = _k23(degp, a1, hs1, W2, b2)
    a2 = _agg_pass(src_r, dst_r, hs2, zini)
    hs3 = _k23(degp, a2, hs2, Wml, bml)
    a3 = _agg_pass(src_r, dst_r, hs3, zini)
    out = _k4(degp, a3, hs3)
    return out[:n, :64], out[:n, 64:]


# speedup vs baseline: 28.3610x; 1.0652x over previous
import functools

import jax
import jax.numpy as jnp
from jax import lax
from jax.experimental import pallas as pl
from jax.experimental.pallas import tpu as pltpu
from jax.experimental.pallas import tpu_sc as plsc

N = 10000
F = 128
NC = 2
NS = 16
NW = NC * NS
CHUNK = 128
NCHUNK = 80
EPT = NCHUNK * CHUNK
EPAD = NW * EPT
NPAD = 10240
STRIPE = NPAD // NS
BR = 1280
GRID = NPAD // BR

_mesh = plsc.VectorSubcoreMesh(core_axis_name="c", subcore_axis_name="s")


@functools.partial(
    pl.kernel,
    out_type=jax.ShapeDtypeStruct((NC, NPAD, F), jnp.float32),
    mesh=_mesh,
    scratch_types=[
        pltpu.VMEM((CHUNK,), jnp.int32),
        pltpu.VMEM((CHUNK,), jnp.int32),
        pltpu.VMEM((CHUNK, F), jnp.float32),
        pltpu.VMEM_SHARED((NPAD, F), jnp.float32),
        pltpu.SemaphoreType.DMA,
        pltpu.SemaphoreType.DMA,
    ],
)
def _deg_pass(dst_hbm, eye_hbm, z_hbm, out_hbm, idx0, idx1, upd_v, deg_sh,
              semi0, semi1):
    c = lax.axis_index("c")
    s = lax.axis_index("s")
    wid = s * NC + c
    pltpu.sync_copy(eye_hbm, upd_v)
    pltpu.sync_copy(z_hbm.at[pl.ds(s * STRIPE, STRIPE)],
                    deg_sh.at[pl.ds(s * STRIPE, STRIPE)])
    plsc.subcore_barrier()

    idx = (idx0, idx1)
    semi = (semi0, semi1)

    def issue(j, b):
        pltpu.async_copy(dst_hbm.at[wid, j], idx[b], semi[b])

    def wait_and_scatter(j, b):
        pltpu.make_async_copy(dst_hbm.at[wid, j], idx[b], semi[b]).wait()
        pltpu.sync_copy(upd_v, deg_sh.at[idx[b]], add=True)

    issue(0, 0)

    def body(jj, carry):
        j0 = jj * 2
        issue(j0 + 1, 1)
        wait_and_scatter(j0, 0)

        @pl.when(j0 + 2 < NCHUNK)
        def _():
            issue(j0 + 2, 0)

        wait_and_scatter(j0 + 1, 1)
        return carry

    lax.fori_loop(0, NCHUNK // 2, body, 0)
    plsc.subcore_barrier()
    pltpu.sync_copy(deg_sh.at[pl.ds(s * STRIPE, STRIPE)],
                    out_hbm.at[c, pl.ds(s * STRIPE, STRIPE)])


@functools.partial(
    pl.kernel,
    out_type=jax.ShapeDtypeStruct((NC, NPAD, F), jnp.float32),
    mesh=_mesh,
    scratch_types=[
        pltpu.VMEM((NCHUNK, CHUNK), jnp.int32),
        pltpu.VMEM((CHUNK,), jnp.int32),
        pltpu.VMEM((CHUNK,), jnp.int32),
        pltpu.VMEM((CHUNK, F), jnp.float32),
        pltpu.VMEM((CHUNK, F), jnp.float32),
        pltpu.VMEM_SHARED((NPAD, F), jnp.float32),
        pltpu.SemaphoreType.DMA,
        pltpu.SemaphoreType.DMA,
        pltpu.SemaphoreType.DMA,
        pltpu.SemaphoreType.DMA,
    ],
)
def _agg_pass(src_hbm, dst_hbm, hs_hbm, z_hbm, out_hbm,
              src_v, didx0, didx1, rows0, rows1, acc_sh,
              semg0, semg1, semi0, semi1):
    c = lax.axis_index("c")
    s = lax.axis_index("s")
    wid = s * NC + c
    pltpu.sync_copy(src_hbm.at[wid], src_v)
    pltpu.sync_copy(z_hbm.at[pl.ds(s * STRIPE, STRIPE)],
                    acc_sh.at[pl.ds(s * STRIPE, STRIPE)])
    plsc.subcore_barrier()

    didx = (didx0, didx1)
    rows = (rows0, rows1)
    semg = (semg0, semg1)
    semi = (semi0, semi1)

    def issue(j, b):
        pltpu.async_copy(dst_hbm.at[wid, j], didx[b], semi[b])
        pltpu.async_copy(hs_hbm.at[src_v.at[j]], rows[b], semg[b])

    def wait_and_scatter(j, b):
        pltpu.make_async_copy(dst_hbm.at[wid, j], didx[b], semi[b]).wait()
        pltpu.make_async_copy(hs_hbm.at[src_v.at[j]], rows[b], semg[b]).wait()
        pltpu.sync_copy(rows[b], acc_sh.at[didx[b]], add=True)

    issue(0, 0)

    def body(jj, carry):
        j0 = jj * 2
        issue(j0 + 1, 1)
        wait_and_scatter(j0, 0)

        @pl.when(j0 + 2 < NCHUNK)
        def _():
            issue(j0 + 2, 0)

        wait_and_scatter(j0 + 1, 1)
        return carry

    lax.fori_loop(0, NCHUNK // 2, body, 0)
    plsc.subcore_barrier()
    pltpu.sync_copy(acc_sh.at[pl.ds(s * STRIPE, STRIPE)],
                    out_hbm.at[c, pl.ds(s * STRIPE, STRIPE)])


def _dinv(degp):
    return lax.rsqrt(1.0 + degp[0, :, 0:1] + degp[1, :, 0:1])


def _k1_body(degp, x, w, b, o):
    dinv = _dinv(degp)
    lin = jnp.dot(x[...], w[...], preferred_element_type=jnp.float32)
    o[...] = (lin + b[...][None, :]) * dinv


def _k23_body(degp, a, hs, w, b, o):
    dinv = _dinv(degp)
    h = jnp.maximum((a[0] + a[1] + hs[...]) * dinv, 0.0)
    lin = jnp.dot(h, w[...], preferred_element_type=jnp.float32)
    o[...] = (lin + b[...][None, :]) * dinv


def _k4_body(degp, a, hs, o):
    o[...] = (a[0] + a[1] + hs[...]) * _dinv(degp)


_spec_degp = pl.BlockSpec((NC, BR, F), lambda i: (0, i, 0))
_spec_rows = pl.BlockSpec((BR, F), lambda i: (i, 0))
_spec_acc = pl.BlockSpec((NC, BR, F), lambda i: (0, i, 0))
_spec_w = pl.BlockSpec((F, F), lambda i: (0, 0))
_spec_b = pl.BlockSpec((F,), lambda i: (0,))
_out_rows = jax.ShapeDtypeStruct((NPAD, F), jnp.float32)

_k1 = pl.pallas_call(
    _k1_body, grid=(GRID,),
    in_specs=[_spec_degp, _spec_rows, _spec_w, _spec_b],
    out_specs=_spec_rows, out_shape=_out_rows)

_k23 = pl.pallas_call(
    _k23_body, grid=(GRID,),
    in_specs=[_spec_degp, _spec_acc, _spec_rows, _spec_w, _spec_b],
    out_specs=_spec_rows, out_shape=_out_rows)

_k4 = pl.pallas_call(
    _k4_body, grid=(GRID,),
    in_specs=[_spec_degp, _spec_acc, _spec_rows],
    out_specs=_spec_rows, out_shape=_out_rows)


def kernel(x, edge_index, W1, b1, W2, b2, Wmu, bmu, Wls, bls):
    n, f = x.shape
    e = edge_index.shape[1]
    src = edge_index[0].astype(jnp.int32)
    dst = edge_index[1].astype(jnp.int32)
    pad = EPAD - e
    ar = jnp.arange(pad, dtype=jnp.int32)
    pad_src = (ar * 131) % n
    pad_dst = n + ar % (NPAD - n)
    src_r = jnp.concatenate([src, pad_src]).reshape(NW, NCHUNK, CHUNK)
    dst_r = jnp.concatenate([dst, pad_dst]).reshape(NW, NCHUNK, CHUNK)
    xp = jnp.pad(x, ((0, NPAD - n), (0, 0)))
    Wml = jnp.concatenate([Wmu, Wls], axis=1)
    bml = jnp.concatenate([bmu, bls])
    zini = jnp.zeros((NPAD, F), jnp.float32)
    eye0 = jnp.zeros((CHUNK, F), jnp.float32).at[:, 0].set(1.0)

    degp = _deg_pass(dst_r, eye0, zini)

    hs1 = _k1(degp, xp, W1, b1)
    a1 = _agg_pass(src_r, dst_r, hs1, zini)
    hs2 = _k23(degp, a1, hs1, W2, b2)
    a2 = _agg_pass(src_r, dst_r, hs2, zini)
    hs3 = _k23(degp, a2, hs2, Wml, bml)
    a3 = _agg_pass(src_r, dst_r, hs3, zini)
    out = _k4(degp, a3, hs3)
    return out[:n, :64], out[:n, 64:]
